# Initial kernel scaffold; baseline (speedup 1.0000x reference)
import jax
import jax.numpy as jnp
from jax.experimental import pallas as pl


def kernel(input_ids, r_table, g_table, b_table):
    raise NotImplementedError("write your pallas kernel here")


# speedup vs baseline: 2.9173x; 2.9173x over previous
import functools

import jax
import jax.numpy as jnp
from jax import lax
from jax.experimental import pallas as pl
from jax.experimental.pallas import tpu as pltpu
from jax.experimental.pallas import tpu_sc as plsc

_DIM = 128
_CHUNK = 128


@functools.lru_cache(maxsize=None)
def _make_gather(n_rows: int):
    info = plsc.get_sparse_core_info()
    nc, ns = info.num_cores, info.num_subcores
    nw = nc * ns
    rows_per_w = n_rows // nw
    assert rows_per_w % _CHUNK == 0
    chunks_per_w = rows_per_w // _CHUNK

    mesh = plsc.VectorSubcoreMesh(core_axis_name="c", subcore_axis_name="s")

    @functools.partial(
        pl.kernel,
        mesh=mesh,
        out_type=jax.ShapeDtypeStruct((n_rows, _DIM), jnp.float32),
        scratch_types=[
            pltpu.VMEM((chunks_per_w, _CHUNK), jnp.int32),
            pltpu.VMEM((_CHUNK, _DIM), jnp.float32),
            pltpu.SemaphoreType.DMA,
        ],
    )
    def gather(table_hbm, idx_hbm, out_hbm, idx_v, rows_v, gsem):
        wid = lax.axis_index("s") * nc + lax.axis_index("c")
        pltpu.sync_copy(idx_hbm.at[pl.ds(wid * chunks_per_w, chunks_per_w)],
                        idx_v)

        def chunk_body(t, carry):
            row0 = (wid * chunks_per_w + t) * _CHUNK
            pltpu.async_copy(table_hbm.at[idx_v.at[t]], rows_v, gsem).wait()
            pltpu.sync_copy(rows_v, out_hbm.at[pl.ds(row0, _CHUNK)])
            return carry

        lax.fori_loop(0, chunks_per_w, chunk_body, 0)

    return gather


def kernel(input_ids, r_table, g_table, b_table):
    assert input_ids.ndim == 3 and input_ids.shape[-1] == 3
    b, s, _ = input_ids.shape
    v = r_table.shape[0]
    table = jnp.concatenate([r_table, g_table, b_table], axis=0)
    idx = input_ids.astype(jnp.int32) + jnp.arange(3, dtype=jnp.int32) * v
    n_rows = b * s * 3
    idx2d = idx.reshape(n_rows // _CHUNK, _CHUNK)
    out = _make_gather(n_rows)(table, idx2d)
    return out.reshape(b, s, 3 * _DIM)


# speedup vs baseline: 3.0559x; 1.0475x over previous
import functools

import jax
import jax.numpy as jnp
from jax import lax
from jax.experimental import pallas as pl
from jax.experimental.pallas import tpu as pltpu
from jax.experimental.pallas import tpu_sc as plsc

_DIM = 128
_CHUNK = 128


@functools.lru_cache(maxsize=None)
def _make_gather(n_rows: int):
    info = plsc.get_sparse_core_info()
    nc, ns = info.num_cores, info.num_subcores
    nw = nc * ns
    rows_per_w = n_rows // nw
    assert rows_per_w % _CHUNK == 0
    chunks_per_w = rows_per_w // _CHUNK

    mesh = plsc.VectorSubcoreMesh(core_axis_name="c", subcore_axis_name="s")

    @functools.partial(
        pl.kernel,
        mesh=mesh,
        out_type=jax.ShapeDtypeStruct((n_rows, _DIM), jnp.float32),
        scratch_types=[
            pltpu.VMEM((chunks_per_w, _CHUNK), jnp.int32),
            pltpu.VMEM((_CHUNK, _DIM), jnp.float32),
            pltpu.VMEM((_CHUNK, _DIM), jnp.float32),
            pltpu.SemaphoreType.DMA,
            pltpu.SemaphoreType.DMA,
            pltpu.SemaphoreType.DMA,
            pltpu.SemaphoreType.DMA,
        ],
    )
    def gather(table_hbm, idx_hbm, out_hbm, idx_v,
               rows0, rows1, gsem0, gsem1, ssem0, ssem1):
        wid = lax.axis_index("s") * nc + lax.axis_index("c")
        base = wid * chunks_per_w
        pltpu.sync_copy(idx_hbm.at[pl.ds(base, chunks_per_w)], idx_v)

        def gstart(t, rows, sem):
            pltpu.async_copy(table_hbm.at[idx_v.at[t]], rows, sem)

        def gwait(t, rows, sem):
            pltpu.make_async_copy(table_hbm.at[idx_v.at[t]], rows, sem).wait()

        def sstart(t, rows, sem):
            pltpu.async_copy(rows, out_hbm.at[pl.ds((base + t) * _CHUNK,
                                                    _CHUNK)], sem)

        def swait(t, rows, sem):
            pltpu.make_async_copy(
                rows, out_hbm.at[pl.ds((base + t) * _CHUNK, _CHUNK)],
                sem).wait()

        bufs = ((rows0, gsem0, ssem0), (rows1, gsem1, ssem1))

        gstart(0, rows0, gsem0)
        gstart(1, rows1, gsem1)

        def pair_body(j, carry):
            c = j * 2
            for b in range(2):
                rows, gsem, ssem = bufs[b]
                gwait(c + b, rows, gsem)
                sstart(c + b, rows, ssem)
                swait(c + b, rows, ssem)
                gstart(c + b + 2, rows, gsem)
            return carry

        lax.fori_loop(0, chunks_per_w // 2 - 1, pair_body, 0)
        for b in range(2):
            c = chunks_per_w - 2 + b
            rows, gsem, ssem = bufs[b]
            gwait(c, rows, gsem)
            sstart(c, rows, ssem)
            swait(c, rows, ssem)

    return gather


def kernel(input_ids, r_table, g_table, b_table):
    assert input_ids.ndim == 3 and input_ids.shape[-1] == 3
    b, s, _ = input_ids.shape
    v = r_table.shape[0]
    table = jnp.concatenate([r_table, g_table, b_table], axis=0)
    idx = input_ids.astype(jnp.int32) + jnp.arange(3, dtype=jnp.int32) * v
    n_rows = b * s * 3
    idx2d = idx.reshape(n_rows // _CHUNK, _CHUNK)
    out = _make_gather(n_rows)(table, idx2d)
    return out.reshape(b, s, 3 * _DIM)


# speedup vs baseline: 7.6520x; 2.5041x over previous
import functools

import jax
import jax.numpy as jnp
from jax import lax
from jax.experimental import pallas as pl
from jax.experimental.pallas import tpu as pltpu
from jax.experimental.pallas import tpu_sc as plsc

_DIM = 128
_CHUNK = 128


@functools.lru_cache(maxsize=None)
def _make_gather(n_pos: int, vocab: int):
    info = plsc.get_sparse_core_info()
    nc, ns = info.num_cores, info.num_subcores
    nw = nc * ns
    pos_per_w = n_pos // nw
    assert pos_per_w % _CHUNK == 0
    chunks_per_w = pos_per_w // _CHUNK
    n_items = 3 * chunks_per_w

    mesh = plsc.VectorSubcoreMesh(core_axis_name="c", subcore_axis_name="s")

    @functools.partial(
        pl.kernel,
        mesh=mesh,
        out_type=jax.ShapeDtypeStruct((n_pos, 3 * _DIM), jnp.float32),
        scratch_types=[
            pltpu.VMEM((chunks_per_w, _CHUNK), jnp.int32),
            pltpu.VMEM((chunks_per_w, _CHUNK), jnp.int32),
            pltpu.VMEM((chunks_per_w, _CHUNK), jnp.int32),
            pltpu.VMEM((_CHUNK, _DIM), jnp.float32),
            pltpu.VMEM((_CHUNK, _DIM), jnp.float32),
            pltpu.SemaphoreType.DMA,
            pltpu.SemaphoreType.DMA,
            pltpu.SemaphoreType.DMA,
            pltpu.SemaphoreType.DMA,
        ],
    )
    def gather(rt, gt, bt, idx_hbm, out_hbm,
               idx0, idx1, idx2, rows0, rows1,
               gsem0, gsem1, ssem0, ssem1):
        wid = lax.axis_index("s") * nc + lax.axis_index("c")
        tables = (rt, gt, bt)
        idxs = (idx0, idx1, idx2)
        for k in range(3):
            pltpu.sync_copy(idx_hbm.at[k, wid], idxs[k])

        bufs = ((rows0, gsem0, ssem0), (rows1, gsem1, ssem1))

        def gstart(t, k, b):
            rows, gsem, _ = bufs[b]
            pltpu.async_copy(tables[k].at[idxs[k].at[t]], rows, gsem)

        def gwait(t, k, b):
            rows, gsem, _ = bufs[b]
            pltpu.make_async_copy(tables[k].at[idxs[k].at[t]], rows,
                                  gsem).wait()

        def _store_dst(t, k):
            return out_hbm.at[pl.ds((wid * chunks_per_w + t) * _CHUNK,
                                    _CHUNK), pl.ds(k * _DIM, _DIM)]

        def sstart(t, k, b):
            rows, _, ssem = bufs[b]
            pltpu.async_copy(rows, _store_dst(t, k), ssem)

        def swait(t, k, b):
            rows, _, ssem = bufs[b]
            pltpu.make_async_copy(rows, _store_dst(t, k), ssem).wait()

        gstart(0, 0, 0)
        gstart(0, 1, 1)

        def group_body(j, carry):
            for i in range(6):
                t = 2 * j + i // 3
                k, b = i % 3, i % 2
                gwait(t, k, b)
                sstart(t, k, b)
                swait(t, k, b)
                t2 = 2 * j + (i + 2) // 3
                gstart(t2, (i + 2) % 3, b)
            return carry

        lax.fori_loop(0, n_items // 6 - 1, group_body, 0)
        for i in range(6):
            c = n_items - 6 + i
            t, k, b = c // 3, c % 3, c % 2
            gwait(t, k, b)
            sstart(t, k, b)
            swait(t, k, b)
            if c + 2 < n_items:
                c2 = c + 2
                gstart(c2 // 3, c2 % 3, b)

    return gather


def kernel(input_ids, r_table, g_table, b_table):
    assert input_ids.ndim == 3 and input_ids.shape[-1] == 3
    b, s, _ = input_ids.shape
    v = r_table.shape[0]
    n_pos = b * s
    nw = 32
    chunks = n_pos // (nw * _CHUNK)
    ids3 = (input_ids.astype(jnp.int32)
            .transpose(2, 0, 1)
            .reshape(3, nw, chunks, _CHUNK))
    out = _make_gather(n_pos, v)(r_table, g_table, b_table, ids3)
    return out.reshape(b, s, 3 * _DIM)


# speedup vs baseline: 17.0717x; 2.2310x over previous
import functools

import jax
import jax.numpy as jnp
from jax import lax
from jax.experimental import pallas as pl
from jax.experimental.pallas import tpu as pltpu
from jax.experimental.pallas import tpu_sc as plsc

_DIM = 128
_CHUNK = 128


@functools.lru_cache(maxsize=None)
def _make_gather(n_pos: int, vocab: int):
    info = plsc.get_sparse_core_info()
    nc, ns = info.num_cores, info.num_subcores
    nw = nc * ns
    pos_per_w = n_pos // nw
    assert pos_per_w % _CHUNK == 0
    chunks_per_w = pos_per_w // _CHUNK
    n_items = 3 * chunks_per_w

    mesh = plsc.VectorSubcoreMesh(core_axis_name="c", subcore_axis_name="s")

    @functools.partial(
        pl.kernel,
        mesh=mesh,
        out_type=jax.ShapeDtypeStruct((n_pos, 3 * _DIM), jnp.float32),
        scratch_types=[
            pltpu.VMEM((chunks_per_w, _CHUNK), jnp.int32),
            pltpu.VMEM((chunks_per_w, _CHUNK), jnp.int32),
            pltpu.VMEM((chunks_per_w, _CHUNK), jnp.int32),
            pltpu.VMEM((_CHUNK, _DIM), jnp.float32),
            pltpu.VMEM((_CHUNK, _DIM), jnp.float32),
            pltpu.VMEM_SHARED((vocab, _DIM), jnp.float32),
            pltpu.VMEM_SHARED((vocab, _DIM), jnp.float32),
            pltpu.VMEM_SHARED((vocab, _DIM), jnp.float32),
            pltpu.SemaphoreType.DMA,
            pltpu.SemaphoreType.DMA,
            pltpu.SemaphoreType.DMA,
            pltpu.SemaphoreType.DMA,
        ],
    )
    def gather(rt, gt, bt, idx_hbm, out_hbm,
               idx0, idx1, idx2, rows0, rows1,
               tab0, tab1, tab2,
               gsem0, gsem1, ssem0, ssem1):
        sid = lax.axis_index("s")
        wid = sid * nc + lax.axis_index("c")
        tables = (tab0, tab1, tab2)
        idxs = (idx0, idx1, idx2)
        @pl.when(sid == 0)
        def _():
            pltpu.sync_copy(rt, tab0)
            pltpu.sync_copy(gt, tab1)
            pltpu.sync_copy(bt, tab2)
        for k in range(3):
            pltpu.sync_copy(idx_hbm.at[k, wid], idxs[k])
        plsc.subcore_barrier()

        bufs = ((rows0, gsem0, ssem0), (rows1, gsem1, ssem1))

        def gstart(t, k, b):
            rows, gsem, _ = bufs[b]
            pltpu.async_copy(tables[k].at[idxs[k].at[t]], rows, gsem)

        def gwait(t, k, b):
            rows, gsem, _ = bufs[b]
            pltpu.make_async_copy(tables[k].at[idxs[k].at[t]], rows,
                                  gsem).wait()

        def _store_dst(t, k):
            return out_hbm.at[pl.ds((wid * chunks_per_w + t) * _CHUNK,
                                    _CHUNK), pl.ds(k * _DIM, _DIM)]

        def sstart(t, k, b):
            rows, _, ssem = bufs[b]
            pltpu.async_copy(rows, _store_dst(t, k), ssem)

        def swait(t, k, b):
            rows, _, ssem = bufs[b]
            pltpu.make_async_copy(rows, _store_dst(t, k), ssem).wait()

        gstart(0, 0, 0)
        gstart(0, 1, 1)

        def group_body(j, carry):
            for i in range(6):
                t = 2 * j + i // 3
                k, b = i % 3, i % 2
                gwait(t, k, b)
                sstart(t, k, b)
                swait(t, k, b)
                t2 = 2 * j + (i + 2) // 3
                gstart(t2, (i + 2) % 3, b)
            return carry

        lax.fori_loop(0, n_items // 6 - 1, group_body, 0)
        for i in range(6):
            c = n_items - 6 + i
            t, k, b = c // 3, c % 3, c % 2
            gwait(t, k, b)
            sstart(t, k, b)
            swait(t, k, b)
            if c + 2 < n_items:
                c2 = c + 2
                gstart(c2 // 3, c2 % 3, b)

    return gather


def kernel(input_ids, r_table, g_table, b_table):
    assert input_ids.ndim == 3 and input_ids.shape[-1] == 3
    b, s, _ = input_ids.shape
    v = r_table.shape[0]
    n_pos = b * s
    nw = 32
    chunks = n_pos // (nw * _CHUNK)
    ids3 = (input_ids.astype(jnp.int32)
            .transpose(2, 0, 1)
            .reshape(3, nw, chunks, _CHUNK))
    out = _make_gather(n_pos, v)(r_table, g_table, b_table, ids3)
    return out.reshape(b, s, 3 * _DIM)


# speedup vs baseline: 17.3931x; 1.0188x over previous
import functools

import jax
import jax.numpy as jnp
from jax import lax
from jax.experimental import pallas as pl
from jax.experimental.pallas import tpu as pltpu
from jax.experimental.pallas import tpu_sc as plsc

_DIM = 128
_CHUNK = 128


@functools.lru_cache(maxsize=None)
def _make_gather(n_pos: int, vocab: int):
    info = plsc.get_sparse_core_info()
    nc, ns = info.num_cores, info.num_subcores
    nw = nc * ns
    pos_per_w = n_pos // nw
    assert pos_per_w % _CHUNK == 0
    chunks_per_w = pos_per_w // _CHUNK
    n_items = 3 * chunks_per_w

    mesh = plsc.VectorSubcoreMesh(core_axis_name="c", subcore_axis_name="s")

    @functools.partial(
        pl.kernel,
        mesh=mesh,
        out_type=jax.ShapeDtypeStruct((n_pos, 3 * _DIM), jnp.float32),
        scratch_types=[
            pltpu.VMEM((chunks_per_w, _CHUNK), jnp.int32),
            pltpu.VMEM((chunks_per_w, _CHUNK), jnp.int32),
            pltpu.VMEM((chunks_per_w, _CHUNK), jnp.int32),
            pltpu.VMEM((_CHUNK, _DIM), jnp.float32),
            pltpu.VMEM((_CHUNK, _DIM), jnp.float32),
            pltpu.VMEM((_CHUNK, _DIM), jnp.float32),
            pltpu.VMEM((_CHUNK, _DIM), jnp.float32),
            pltpu.VMEM_SHARED((vocab, _DIM), jnp.float32),
            pltpu.VMEM_SHARED((vocab, _DIM), jnp.float32),
            pltpu.VMEM_SHARED((vocab, _DIM), jnp.float32),
            pltpu.SemaphoreType.DMA,
            pltpu.SemaphoreType.DMA,
            pltpu.SemaphoreType.DMA,
            pltpu.SemaphoreType.DMA,
            pltpu.SemaphoreType.DMA,
            pltpu.SemaphoreType.DMA,
            pltpu.SemaphoreType.DMA,
            pltpu.SemaphoreType.DMA,
        ],
    )
    def gather(rt, gt, bt, idx_hbm, out_hbm,
               idx0, idx1, idx2, rows0, rows1, rows2, rows3,
               tab0, tab1, tab2,
               gsem0, gsem1, gsem2, gsem3, ssem0, ssem1, ssem2, ssem3):
        sid = lax.axis_index("s")
        wid = sid * nc + lax.axis_index("c")
        tables = (tab0, tab1, tab2)
        idxs = (idx0, idx1, idx2)
        @pl.when(sid == 0)
        def _():
            pltpu.sync_copy(rt, tab0)
            pltpu.sync_copy(gt, tab1)
            pltpu.sync_copy(bt, tab2)
        for k in range(3):
            pltpu.sync_copy(idx_hbm.at[k, wid], idxs[k])
        plsc.subcore_barrier()

        bufs = ((rows0, gsem0, ssem0), (rows1, gsem1, ssem1),
                (rows2, gsem2, ssem2), (rows3, gsem3, ssem3))
        nbuf = 4

        def gstart(t, k, b):
            rows, gsem, _ = bufs[b]
            pltpu.async_copy(tables[k].at[idxs[k].at[t]], rows, gsem)

        def gwait(t, k, b):
            rows, gsem, _ = bufs[b]
            pltpu.make_async_copy(tables[k].at[idxs[k].at[t]], rows,
                                  gsem).wait()

        def _store_dst(t, k):
            return out_hbm.at[pl.ds((wid * chunks_per_w + t) * _CHUNK,
                                    _CHUNK), pl.ds(k * _DIM, _DIM)]

        def sstart(t, k, b):
            rows, _, ssem = bufs[b]
            pltpu.async_copy(rows, _store_dst(t, k), ssem)

        def swait(t, k, b):
            rows, _, ssem = bufs[b]
            pltpu.make_async_copy(rows, _store_dst(t, k), ssem).wait()

        gstart(0, 0, 0)
        gstart(0, 1, 1)
        gstart(0, 2, 2)

        def full_item(j, i, first_group=False, last_group=False):
            c_t = 4 * j + i // 3
            k, b = i % 3, i % 4
            gwait(c_t, k, b)
            sstart(c_t, k, b)
            if not (first_group and i == 0):
                ip = i - 1 if i >= 1 else 11
                jp = j if i >= 1 else j - 1
                swait(4 * jp + ip // 3, ip % 3, ip % 4)
            if not (last_group and i >= 9):
                i3, j3 = (i + 3) % 12, j + (i + 3) // 12
                gstart(4 * j3 + i3 // 3, i3 % 3, i3 % 4)

        for i in range(12):
            full_item(0, i, first_group=True)

        def group_body(j, carry):
            for i in range(12):
                full_item(j, i)
            return carry

        n_groups = n_items // 12
        lax.fori_loop(1, n_groups - 1, group_body, 0)
        for i in range(12):
            full_item(n_groups - 1, i, last_group=True)
        c = n_items - 1
        swait(c // 3, c % 3, c % 4)

    return gather


def kernel(input_ids, r_table, g_table, b_table):
    assert input_ids.ndim == 3 and input_ids.shape[-1] == 3
    b, s, _ = input_ids.shape
    v = r_table.shape[0]
    n_pos = b * s
    nw = 32
    chunks = n_pos // (nw * _CHUNK)
    ids3 = (input_ids.astype(jnp.int32)
            .transpose(2, 0, 1)
            .reshape(3, nw, chunks, _CHUNK))
    out = _make_gather(n_pos, v)(r_table, g_table, b_table, ids3)
    return out.reshape(b, s, 3 * _DIM)


# speedup vs baseline: 17.3936x; 1.0000x over previous
import functools

import jax
import jax.numpy as jnp
from jax import lax
from jax.experimental import pallas as pl
from jax.experimental.pallas import tpu as pltpu
from jax.experimental.pallas import tpu_sc as plsc

_DIM = 128
_CHUNK = 128


@functools.lru_cache(maxsize=None)
def _make_gather(n_pos: int, vocab: int):
    info = plsc.get_sparse_core_info()
    nc, ns = info.num_cores, info.num_subcores
    nw = nc * ns
    pos_per_w = n_pos // nw
    assert pos_per_w % _CHUNK == 0
    chunks_per_w = pos_per_w // _CHUNK
    n_items = 3 * chunks_per_w

    mesh = plsc.VectorSubcoreMesh(core_axis_name="c", subcore_axis_name="s")

    @functools.partial(
        pl.kernel,
        mesh=mesh,
        out_type=jax.ShapeDtypeStruct((n_pos, 3 * _DIM), jnp.float32),
        scratch_types=[
            pltpu.VMEM((chunks_per_w, _CHUNK), jnp.int32),
            pltpu.VMEM((chunks_per_w, _CHUNK), jnp.int32),
            pltpu.VMEM((chunks_per_w, _CHUNK), jnp.int32),
            pltpu.VMEM((_CHUNK, _DIM), jnp.float32),
            pltpu.VMEM((_CHUNK, _DIM), jnp.float32),
            pltpu.VMEM((_CHUNK, _DIM), jnp.float32),
            pltpu.VMEM((_CHUNK, _DIM), jnp.float32),
            pltpu.VMEM_SHARED((vocab, _DIM), jnp.float32),
            pltpu.VMEM_SHARED((vocab, _DIM), jnp.float32),
            pltpu.VMEM_SHARED((vocab, _DIM), jnp.float32),
            pltpu.SemaphoreType.DMA,
            pltpu.SemaphoreType.DMA,
            pltpu.SemaphoreType.DMA,
            pltpu.SemaphoreType.DMA,
            pltpu.SemaphoreType.DMA,
            pltpu.SemaphoreType.DMA,
            pltpu.SemaphoreType.DMA,
            pltpu.SemaphoreType.DMA,
        ],
    )
    def gather(rt, gt, bt, idx_hbm, out_hbm,
               idx0, idx1, idx2, rows0, rows1, rows2, rows3,
               tab0, tab1, tab2,
               gsem0, gsem1, gsem2, gsem3, ssem0, ssem1, ssem2, ssem3):
        sid = lax.axis_index("s")
        wid = sid * nc + lax.axis_index("c")
        tables = (tab0, tab1, tab2)
        idxs = (idx0, idx1, idx2)
        rows_per_sub = vocab // ns
        stage = pl.ds(sid * rows_per_sub, rows_per_sub)
        pltpu.sync_copy(rt.at[stage], tab0.at[stage])
        pltpu.sync_copy(gt.at[stage], tab1.at[stage])
        pltpu.sync_copy(bt.at[stage], tab2.at[stage])
        for k in range(3):
            pltpu.sync_copy(idx_hbm.at[k, wid], idxs[k])
        plsc.subcore_barrier()

        bufs = ((rows0, gsem0, ssem0), (rows1, gsem1, ssem1),
                (rows2, gsem2, ssem2), (rows3, gsem3, ssem3))
        nbuf = 4

        def gstart(t, k, b):
            rows, gsem, _ = bufs[b]
            pltpu.async_copy(tables[k].at[idxs[k].at[t]], rows, gsem)

        def gwait(t, k, b):
            rows, gsem, _ = bufs[b]
            pltpu.make_async_copy(tables[k].at[idxs[k].at[t]], rows,
                                  gsem).wait()

        def _store_dst(t, k):
            return out_hbm.at[pl.ds((wid * chunks_per_w + t) * _CHUNK,
                                    _CHUNK), pl.ds(k * _DIM, _DIM)]

        def sstart(t, k, b):
            rows, _, ssem = bufs[b]
            pltpu.async_copy(rows, _store_dst(t, k), ssem)

        def swait(t, k, b):
            rows, _, ssem = bufs[b]
            pltpu.make_async_copy(rows, _store_dst(t, k), ssem).wait()

        gstart(0, 0, 0)
        gstart(0, 1, 1)
        gstart(0, 2, 2)

        def full_item(j, i, first_group=False, last_group=False):
            c_t = 4 * j + i // 3
            k, b = i % 3, i % 4
            gwait(c_t, k, b)
            sstart(c_t, k, b)
            if not (first_group and i == 0):
                ip = i - 1 if i >= 1 else 11
                jp = j if i >= 1 else j - 1
                swait(4 * jp + ip // 3, ip % 3, ip % 4)
            if not (last_group and i >= 9):
                i3, j3 = (i + 3) % 12, j + (i + 3) // 12
                gstart(4 * j3 + i3 // 3, i3 % 3, i3 % 4)

        for i in range(12):
            full_item(0, i, first_group=True)

        def group_body(j, carry):
            for i in range(12):
                full_item(j, i)
            return carry

        n_groups = n_items // 12
        lax.fori_loop(1, n_groups - 1, group_body, 0)
        for i in range(12):
            full_item(n_groups - 1, i, last_group=True)
        c = n_items - 1
        swait(c // 3, c % 3, c % 4)

    return gather


def kernel(input_ids, r_table, g_table, b_table):
    assert input_ids.ndim == 3 and input_ids.shape[-1] == 3
    b, s, _ = input_ids.shape
    v = r_table.shape[0]
    n_pos = b * s
    nw = 32
    chunks = n_pos // (nw * _CHUNK)
    ids3 = (input_ids.astype(jnp.int32)
            .transpose(2, 0, 1)
            .reshape(3, nw, chunks, _CHUNK))
    out = _make_gather(n_pos, v)(r_table, g_table, b_table, ids3)
    return out.reshape(b, s, 3 * _DIM)
